# baseline (speedup 1.0000x reference)
import functools

import jax
import jax.numpy as jnp
from jax import lax
from jax.experimental import pallas as pl
from jax.experimental.pallas import tpu as pltpu
from jax.experimental.pallas import tpu_sc as plsc

_HIDDEN = 768
_EXPERTS = 8
_BLOCK = 4096
_LANES = 16


def _logits_kernel(x_ref, w_ref, b_ref, lt_ref):
    x = x_ref[...]
    w = w_ref[...]
    b = b_ref[...]
    logits = jax.lax.dot_general(
        x, w, (((1,), (1,)), ((), ())),
        preferred_element_type=jnp.float32)
    lt_ref[...] = logits.T + b.T


def _tc_logits(x):
    n = x.shape[0]
    return pl.pallas_call(
        _logits_kernel,
        grid=(n // _BLOCK,),
        in_specs=[
            pl.BlockSpec((_BLOCK, _HIDDEN), lambda i: (i, 0)),
            pl.BlockSpec((_EXPERTS, _HIDDEN), lambda i: (0, 0)),
            pl.BlockSpec((1, _EXPERTS), lambda i: (0, 0)),
        ],
        out_specs=pl.BlockSpec((_EXPERTS, _BLOCK), lambda i: (0, i)),
        out_shape=jax.ShapeDtypeStruct((_EXPERTS, n), jnp.float32),
    )


def _make_sc_router(n):
    info = plsc.get_sparse_core_info()
    nw = info.num_cores * info.num_subcores
    chunk = n // nw
    mesh = plsc.VectorSubcoreMesh(core_axis_name="c", subcore_axis_name="s")

    @functools.partial(
        pl.kernel, mesh=mesh,
        out_type=[
            jax.ShapeDtypeStruct((_EXPERTS, n), jnp.float32),
            jax.ShapeDtypeStruct((2, n), jnp.int32),
        ],
        scratch_types=[
            pltpu.VMEM((_EXPERTS, chunk), jnp.float32),
            pltpu.VMEM((_EXPERTS, chunk), jnp.float32),
            pltpu.VMEM((2, chunk), jnp.int32),
        ],
    )
    def _router(lt_hbm, scores_hbm, idx_hbm, lt_v, sc_v, ix_v):
        wid = lax.axis_index("s") * info.num_cores + lax.axis_index("c")
        base = wid * chunk
        pltpu.sync_copy(lt_hbm.at[:, pl.ds(base, chunk)], lt_v)

        def body(g, carry):
            o = g * _LANES
            sl = pl.ds(o, _LANES)
            m1 = lt_v[0, sl]
            a1 = jnp.zeros((_LANES,), jnp.int32)
            for e in range(1, _EXPERTS):
                v = lt_v[e, sl]
                upd = v > m1
                m1 = jnp.where(upd, v, m1)
                a1 = jnp.where(upd, e, a1)
            m2 = jnp.full((_LANES,), -jnp.inf, jnp.float32)
            a2 = jnp.zeros((_LANES,), jnp.int32)
            for e in range(_EXPERTS):
                v = jnp.where(a1 == e, -jnp.inf, lt_v[e, sl])
                upd = v > m2
                m2 = jnp.where(upd, v, m2)
                a2 = jnp.where(upd, e, a2)
            e2 = jnp.exp(m2 - m1)
            denom = 1.0 + e2
            p1 = 1.0 / denom
            p2 = e2 / denom
            for e in range(_EXPERTS):
                sc_v[e, sl] = jnp.where(a1 == e, p1,
                                        jnp.where(a2 == e, p2, 0.0))
            ix_v[0, sl] = a1
            ix_v[1, sl] = a2
            return carry

        lax.fori_loop(0, chunk // _LANES, body, 0)
        pltpu.sync_copy(sc_v, scores_hbm.at[:, pl.ds(base, chunk)])
        pltpu.sync_copy(ix_v, idx_hbm.at[:, pl.ds(base, chunk)])

    return _router


def kernel(hidden_states, weight, bias):
    x = hidden_states.reshape(-1, _HIDDEN)
    n = x.shape[0]
    lt = _tc_logits(x)(x, weight, bias.reshape(1, _EXPERTS))
    scores_t, idx_t = _make_sc_router(n)(lt)
    return scores_t.T, idx_t.T


# speedup vs baseline: 1.3489x; 1.3489x over previous
import jax
import jax.numpy as jnp
from jax.experimental import pallas as pl

_HIDDEN = 768
_EXPERTS = 8
_BLOCK = 4096


def _route(lt, scores_ref, idx_ref):
    e = jax.lax.broadcasted_iota(jnp.int32, lt.shape, 0)
    m1 = jnp.max(lt, axis=0, keepdims=True)
    a1 = jnp.min(jnp.where(lt == m1, e, _EXPERTS), axis=0, keepdims=True)
    masked = jnp.where(e == a1, -jnp.inf, lt)
    m2 = jnp.max(masked, axis=0, keepdims=True)
    a2 = jnp.min(jnp.where(masked == m2, e, _EXPERTS), axis=0, keepdims=True)

    e2 = jnp.exp(m2 - m1)
    denom = 1.0 + e2
    p1 = 1.0 / denom
    p2 = e2 / denom

    scores_ref[...] = jnp.where(e == a1, p1, jnp.where(e == a2, p2, 0.0))
    idx_ref[...] = jnp.concatenate([a1, a2], axis=0)


def _router_kernel(xa_ref, xb_ref, w_ref, b_ref, sa_ref, ia_ref, sb_ref, ib_ref):
    w = w_ref[...]
    b = b_ref[...]
    dn = (((1,), (1,)), ((), ()))
    la = jax.lax.dot_general(xa_ref[...], w, dn,
                             preferred_element_type=jnp.float32)
    _route(la.T + b.T, sa_ref, ia_ref)
    lb = jax.lax.dot_general(xb_ref[...], w, dn,
                             preferred_element_type=jnp.float32)
    _route(lb.T + b.T, sb_ref, ib_ref)


def kernel(hidden_states, weight, bias):
    x = hidden_states.reshape(-1, _HIDDEN)
    n = x.shape[0]
    half = n // 2
    steps = half // _BLOCK
    off = steps

    sa, ia, sb, ib = pl.pallas_call(
        _router_kernel,
        grid=(steps,),
        in_specs=[
            pl.BlockSpec((_BLOCK, _HIDDEN), lambda i: (i, 0)),
            pl.BlockSpec((_BLOCK, _HIDDEN), lambda i: (i + off, 0)),
            pl.BlockSpec((_EXPERTS, _HIDDEN), lambda i: (0, 0)),
            pl.BlockSpec((1, _EXPERTS), lambda i: (0, 0)),
        ],
        out_specs=[
            pl.BlockSpec((_EXPERTS, _BLOCK), lambda i: (0, i)),
            pl.BlockSpec((2, _BLOCK), lambda i: (0, i)),
            pl.BlockSpec((_EXPERTS, _BLOCK), lambda i: (0, i)),
            pl.BlockSpec((2, _BLOCK), lambda i: (0, i)),
        ],
        out_shape=[
            jax.ShapeDtypeStruct((_EXPERTS, half), jnp.float32),
            jax.ShapeDtypeStruct((2, half), jnp.int32),
            jax.ShapeDtypeStruct((_EXPERTS, half), jnp.float32),
            jax.ShapeDtypeStruct((2, half), jnp.int32),
        ],
    )(x, x, weight, bias.reshape(1, _EXPERTS))
    scores_t = jnp.concatenate([sa, sb], axis=1)
    idx_t = jnp.concatenate([ia, ib], axis=1)
    return scores_t.T, idx_t.T


# speedup vs baseline: 1.6098x; 1.1934x over previous
import jax
import jax.numpy as jnp
from jax.experimental import pallas as pl

_HIDDEN = 768
_EXPERTS = 8
_BLOCK = 4096


def _router_kernel(x_ref, w_ref, b_ref, scores_ref, idx_ref):
    x = x_ref[...]
    w = w_ref[...]
    b = b_ref[...]
    logits = jax.lax.dot_general(
        x, w, (((1,), (1,)), ((), ())),
        preferred_element_type=jnp.float32)
    lt = logits.T + b.T

    e = jax.lax.broadcasted_iota(jnp.int32, lt.shape, 0)
    m1 = jnp.max(lt, axis=0, keepdims=True)
    a1 = jnp.min(jnp.where(lt == m1, e, _EXPERTS), axis=0, keepdims=True)
    masked = jnp.where(e == a1, -jnp.inf, lt)
    m2 = jnp.max(masked, axis=0, keepdims=True)
    a2 = jnp.min(jnp.where(masked == m2, e, _EXPERTS), axis=0, keepdims=True)

    e2 = jnp.exp(m2 - m1)
    denom = 1.0 + e2
    p1 = 1.0 / denom
    p2 = e2 / denom

    scores_ref[...] = jnp.where(e == a1, p1, jnp.where(e == a2, p2, 0.0))
    idx_ref[...] = jnp.concatenate([a1, a2], axis=0)


def kernel(hidden_states, weight, bias):
    x = hidden_states.reshape(-1, _HIDDEN)
    n = x.shape[0]
    grid = (n // _BLOCK,)
    scores_t, idx_t = pl.pallas_call(
        _router_kernel,
        grid=grid,
        in_specs=[
            pl.BlockSpec((_BLOCK, _HIDDEN), lambda i: (i, 0)),
            pl.BlockSpec((_EXPERTS, _HIDDEN), lambda i: (0, 0)),
            pl.BlockSpec((1, _EXPERTS), lambda i: (0, 0)),
        ],
        out_specs=[
            pl.BlockSpec((_EXPERTS, _BLOCK), lambda i: (0, i)),
            pl.BlockSpec((2, _BLOCK), lambda i: (0, i)),
        ],
        out_shape=[
            jax.ShapeDtypeStruct((_EXPERTS, n), jnp.float32),
            jax.ShapeDtypeStruct((2, n), jnp.int32),
        ],
    )(x, weight, bias.reshape(1, _EXPERTS))
    return scores_t.T, idx_t.T
